# baseline (speedup 1.0000x reference)
import jax
import jax.numpy as jnp
from jax.experimental import pallas as pl
from jax.experimental.pallas import tpu as pltpu

N = 8388608
ROWS = 65536
COLS = 128
BLOCK_ROWS = 8192
GRID = ROWS // BLOCK_ROWS


def _fused_body(p_ref, t_ref, mix_ref, t_sum_ref):
    i = pl.program_id(0)

    p = p_ref[...]
    t = t_ref[...]

    LN2 = 0.6931471805599453
    NLOG2E = -1.4426950408889634
    y = jnp.minimum(p * NLOG2E, 126.0)
    e = jnp.exp2(y)
    s = LN2 * jnp.log2(1.0 + e)
    u = LN2 * jnp.log2(1.0 + s)
    mix = jnp.where(t >= 0.5, -100.0, u)

    mix_part = jnp.sum(mix.reshape(BLOCK_ROWS // 8, 8, COLS), axis=0)
    t_part = jnp.sum(t.reshape(BLOCK_ROWS // 8, 8, COLS), axis=0)

    @pl.when(i == 0)
    def _():
        mix_ref[...] = jnp.zeros_like(mix_ref)
        t_sum_ref[...] = jnp.zeros_like(t_sum_ref)

    mix_ref[...] += mix_part
    t_sum_ref[...] += t_part


def kernel(pred, target):
    p2 = pred.reshape(ROWS, COLS)
    t2 = target.reshape(ROWS, COLS)

    mix_acc, t_acc = pl.pallas_call(
        _fused_body,
        grid=(GRID,),
        in_specs=[
            pl.BlockSpec((BLOCK_ROWS, COLS), lambda i: (i, 0)),
            pl.BlockSpec((BLOCK_ROWS, COLS), lambda i: (i, 0)),
        ],
        out_specs=[
            pl.BlockSpec((8, COLS), lambda i: (0, 0)),
            pl.BlockSpec((8, COLS), lambda i: (0, 0)),
        ],
        out_shape=[
            jax.ShapeDtypeStruct((8, COLS), jnp.float32),
            jax.ShapeDtypeStruct((8, COLS), jnp.float32),
        ],
    )(p2, t2)

    s_mix = jnp.sum(mix_acc)
    s_t = jnp.sum(t_acc)

    a = jnp.float32(N) - s_t
    b = s_t
    w = a / b
    return -(w * s_mix) / jnp.float32(N)
